# baseline (speedup 1.0000x reference)
import functools

import jax
import jax.numpy as jnp
from jax import lax
from jax.experimental import pallas as pl
from jax.experimental.pallas import tpu as pltpu
from jax.experimental.pallas import tpu_sc as plsc

_EMBED = 64
_LANES = 16


def _dot_body(pe_t, ue_t, idx, out, idx_v, colp, colu, res, sem_p, sem_u):
    c = lax.axis_index("c")
    s = lax.axis_index("s")

    @pl.when((c == 0) & (s == 0))
    def _():
        pltpu.sync_copy(idx, idx_v)
        iv = idx_v[...]
        p = iv[0]
        u = iv[1]
        p_blk = pl.multiple_of((p >> 7) << 7, 128)
        u_blk = pl.multiple_of((u >> 7) << 7, 128)
        cp_p = pltpu.async_copy(pe_t.at[:, pl.ds(p_blk, 128)], colp, sem_p)
        cp_u = pltpu.async_copy(ue_t.at[:, pl.ds(u_blk, 128)], colu, sem_u)
        cp_p.wait()
        cp_u.wait()
        p_col = jnp.full((_LANES,), p & 127, jnp.int32)
        u_col = jnp.full((_LANES,), u & 127, jnp.int32)
        lanes16 = lax.iota(jnp.int32, _LANES)
        acc = jnp.zeros((_LANES,), jnp.float32)
        for k in range(_EMBED // _LANES):
            rows = lanes16 + (k * _LANES)
            acc = acc + (plsc.load_gather(colp, [rows, p_col]) *
                         plsc.load_gather(colu, [rows, u_col]))
        lane = lax.iota(jnp.int32, _LANES)
        for shift in (8, 4, 2, 1):
            partner = jnp.bitwise_xor(lane, shift)
            acc = acc + lax.gather(
                acc, partner[:, None],
                lax.GatherDimensionNumbers(offset_dims=(),
                                           collapsed_slice_dims=(0,),
                                           start_index_map=(0,)),
                slice_sizes=(1,),
                mode=lax.GatherScatterMode.PROMISE_IN_BOUNDS)
        res[...] = acc
        pltpu.sync_copy(res, out)


_sc_dot = functools.partial(
    pl.kernel,
    out_type=jax.ShapeDtypeStruct((_LANES,), jnp.float32),
    mesh=plsc.VectorSubcoreMesh(core_axis_name="c", subcore_axis_name="s",
                                num_cores=1),
    compiler_params=pltpu.CompilerParams(needs_layout_passes=False, skip_device_barrier=True),
    scratch_types=[
        pltpu.VMEM((_LANES,), jnp.int32),
        pltpu.VMEM((_EMBED, 128), jnp.float32),
        pltpu.VMEM((_EMBED, 128), jnp.float32),
        pltpu.VMEM((_LANES,), jnp.float32),
        pltpu.SemaphoreType.DMA,
        pltpu.SemaphoreType.DMA,
    ],
)(_dot_body)


def kernel(product_embedding, user_embedding, product, user):
    idx = jnp.stack([jnp.asarray(product, jnp.int32),
                     jnp.asarray(user, jnp.int32)])
    idx = jnp.pad(idx, (0, _LANES - 2))
    out = _sc_dot(product_embedding.T, user_embedding.T, idx)
    return out[0]


# speedup vs baseline: 1.0055x; 1.0055x over previous
import functools

import jax
import jax.numpy as jnp
from jax import lax
from jax.experimental import pallas as pl
from jax.experimental.pallas import tpu as pltpu
from jax.experimental.pallas import tpu_sc as plsc

_EMBED = 64
_LANES = 16


def _dot_body(pe_t, ue_t, idx, out, idx_v, colp, colu, res, sem_p, sem_u):
    c = lax.axis_index("c")
    s = lax.axis_index("s")

    @pl.when((c == 0) & (s == 0))
    def _():
        pltpu.sync_copy(idx, idx_v)
        iv = idx_v[...]
        p = iv[0]
        u = iv[1]
        p_blk = pl.multiple_of((p >> 7) << 7, 128)
        u_blk = pl.multiple_of((u >> 7) << 7, 128)
        cp_p = pltpu.async_copy(pe_t.at[:, pl.ds(p_blk, 128)], colp, sem_p)
        cp_u = pltpu.async_copy(ue_t.at[:, pl.ds(u_blk, 128)], colu, sem_u)
        cp_p.wait()
        cp_u.wait()
        p_col = jnp.full((_LANES,), p & 127, jnp.int32)
        u_col = jnp.full((_LANES,), u & 127, jnp.int32)
        lanes16 = lax.iota(jnp.int32, _LANES)
        acc = jnp.zeros((_LANES,), jnp.float32)
        for k in range(_EMBED // _LANES):
            rows = lanes16 + (k * _LANES)
            acc = acc + (plsc.load_gather(colp, [rows, p_col]) *
                         plsc.load_gather(colu, [rows, u_col]))
        lane = lax.iota(jnp.int32, _LANES)
        for shift in (8, 4, 2, 1):
            partner = jnp.bitwise_xor(lane, shift)
            acc = acc + lax.gather(
                acc, partner[:, None],
                lax.GatherDimensionNumbers(offset_dims=(),
                                           collapsed_slice_dims=(0,),
                                           start_index_map=(0,)),
                slice_sizes=(1,),
                mode=lax.GatherScatterMode.PROMISE_IN_BOUNDS)
        res[...] = acc
        pltpu.sync_copy(res, out)


_sc_dot = functools.partial(
    pl.kernel,
    out_type=jax.ShapeDtypeStruct((_LANES,), jnp.float32),
    mesh=plsc.VectorSubcoreMesh(core_axis_name="c", subcore_axis_name="s",
                                num_cores=1),
    compiler_params=pltpu.CompilerParams(needs_layout_passes=False),
    scratch_types=[
        pltpu.VMEM((_LANES,), jnp.int32),
        pltpu.VMEM((_EMBED, 128), jnp.float32),
        pltpu.VMEM((_EMBED, 128), jnp.float32),
        pltpu.VMEM((_LANES,), jnp.float32),
        pltpu.SemaphoreType.DMA,
        pltpu.SemaphoreType.DMA,
    ],
)(_dot_body)


def kernel(product_embedding, user_embedding, product, user):
    idx = jnp.stack([jnp.asarray(product, jnp.int32),
                     jnp.asarray(user, jnp.int32)])
    idx = jnp.pad(idx, (0, _LANES - 2))
    out = _sc_dot(product_embedding.T, user_embedding.T, idx)
    return out[0]
